# Initial kernel scaffold; baseline (speedup 1.0000x reference)
import jax
import jax.numpy as jnp
from jax.experimental import pallas as pl


def kernel(x, edge_index, edge_attr, batch, W0, b0, W1, b1, W2, b2):
    raise NotImplementedError("write your pallas kernel here")


# speedup vs baseline: 92.0674x; 92.0674x over previous
import functools
import jax
import jax.numpy as jnp
from jax import lax
from jax.experimental import pallas as pl
from jax.experimental.pallas import tpu as pltpu
from jax.experimental.pallas import tpu_sc as plsc

L = 16
NW = 16
NP = 10240


def _make_sc_pipeline(E, n_nodes):
    EW = E // NW
    NSLICE = NP // NW
    VECS_E = EW // L
    VECS_S = NSLICE // L
    assert EW % L == 0 and EW % 8 == 0 and n_nodes <= NP

    mesh = plsc.VectorSubcoreMesh(
        core_axis_name="c", subcore_axis_name="s", num_cores=1)

    @functools.partial(
        pl.kernel,
        mesh=mesh,
        compiler_params=pltpu.CompilerParams(needs_layout_passes=False),
        out_type=jax.ShapeDtypeStruct((E,), jnp.float32),
        scratch_types=[
            pltpu.VMEM((EW,), jnp.int32),
            pltpu.VMEM((EW,), jnp.int32),
            pltpu.VMEM((EW,), jnp.float32),
            pltpu.VMEM((NP,), jnp.float32),
            pltpu.VMEM((NW, NSLICE), jnp.float32),
            pltpu.VMEM((NSLICE,), jnp.float32),
            pltpu.VMEM((NSLICE,), jnp.float32),
            pltpu.VMEM((8, L), jnp.float32),
            pltpu.VMEM_SHARED((NW, NP), jnp.float32),
            pltpu.VMEM_SHARED((NP,), jnp.float32),
        ],
    )
    def sc_pipe(src_hbm, dst_hbm, ea_hbm, pvec_hbm, zeros_hbm, u_hbm,
                src_v, dst_v, h_v, m_v, t16_v, s_v, binv_v, pv_v,
                part_sh, bcast_sh):
        w = lax.axis_index("s")
        base = w * EW
        nbase = w * NSLICE
        pltpu.sync_copy(src_hbm.at[pl.ds(base, EW)], src_v)
        pltpu.sync_copy(dst_hbm.at[pl.ds(base, EW)], dst_v)
        pltpu.sync_copy(ea_hbm.at[pl.ds(base, EW)], h_v)
        pltpu.sync_copy(pvec_hbm, pv_v)

        ones = jnp.full((L,), 1.0, jnp.float32)

        def zero_acc():
            pltpu.sync_copy(zeros_hbm, m_v)

        def scatter(use_h):
            def body(i, carry):
                b = pl.ds(i * L, L)
                si = src_v[b]
                di = dst_v[b]
                v = h_v[b] if use_h else ones
                plsc.addupdate_scatter(m_v, [si], v)
                plsc.addupdate_scatter(m_v, [di], v)
                return carry
            lax.fori_loop(0, VECS_E, body, 0)

        def publish_reduce(is_degree_round):
            pltpu.sync_copy(m_v, part_sh.at[w])
            plsc.subcore_barrier()
            for k in range(NW):
                pltpu.sync_copy(part_sh.at[k, pl.ds(nbase, NSLICE)],
                                t16_v.at[k])

            def rbody(j, carry):
                bj = pl.ds(j * L, L)
                acc = t16_v[0, bj]
                for k in range(1, NW):
                    acc = acc + t16_v[k, bj]
                if is_degree_round:
                    binv_v[bj] = jnp.where(acc > 0.0, 1.0 / acc, 0.0)
                else:
                    s_v[bj] = acc * binv_v[bj]
                return carry
            lax.fori_loop(0, VECS_S, rbody, 0)
            plsc.subcore_barrier()

        def broadcast_m():
            pltpu.sync_copy(s_v, bcast_sh.at[pl.ds(nbase, NSLICE)])
            plsc.subcore_barrier()
            pltpu.sync_copy(bcast_sh, m_v)

        def gather_round(cw, cb, do_relu):
            def body(i, carry):
                b = pl.ds(i * L, L)
                si = src_v[b]
                di = dst_v[b]
                a = plsc.load_gather(m_v, [si])
                c = plsc.load_gather(m_v, [di])
                h = cw * (a + c) + cb
                if do_relu:
                    h = jnp.maximum(h, 0.0)
                h_v[b] = h
                return carry
            lax.fori_loop(0, VECS_E, body, 0)

        zero_acc()
        scatter(use_h=False)
        publish_reduce(is_degree_round=True)

        hw0 = pv_v[0]
        cb0 = pv_v[1]
        hw1 = pv_v[2]
        cb1 = pv_v[3]
        half = pv_v[4]
        zero = pv_v[5]

        zero_acc()
        scatter(use_h=True)
        publish_reduce(is_degree_round=False)
        broadcast_m()
        gather_round(hw0, cb0, True)

        zero_acc()
        scatter(use_h=True)
        publish_reduce(is_degree_round=False)
        broadcast_m()
        gather_round(hw1, cb1, True)

        zero_acc()
        scatter(use_h=True)
        publish_reduce(is_degree_round=False)
        broadcast_m()
        gather_round(half, zero, False)

        pltpu.sync_copy(h_v, u_hbm.at[pl.ds(base, EW)])

    return sc_pipe


def _tc_outer(u_ref, w2_ref, b2_ref, o_ref):
    o_ref[...] = u_ref[...] * w2_ref[...] + b2_ref[...]


def kernel(x, edge_index, edge_attr, batch, W0, b0, W1, b1, W2, b2):
    E = edge_attr.shape[0]
    n_nodes = x.shape[0]
    F = W2.shape[0]

    src = edge_index[0]
    dst = edge_index[1]
    pvec = (jnp.zeros((8, L), jnp.float32)
            .at[0].set(0.5 * W0[0, 0])
            .at[1].set(b0[0])
            .at[2].set(0.5 * W1[0, 0])
            .at[3].set(b1[0])
            .at[4].set(0.5))
    zeros = jnp.zeros((NP,), jnp.float32)

    u = _make_sc_pipeline(E, n_nodes)(src, dst, edge_attr, pvec, zeros)

    BLK = 1280
    out = pl.pallas_call(
        _tc_outer,
        grid=(E // BLK,),
        in_specs=[
            pl.BlockSpec((BLK, 1), lambda i: (i, 0)),
            pl.BlockSpec((1, F), lambda i: (0, 0)),
            pl.BlockSpec((1, F), lambda i: (0, 0)),
        ],
        out_specs=pl.BlockSpec((BLK, F), lambda i: (i, 0)),
        out_shape=jax.ShapeDtypeStruct((E, F), jnp.float32),
        compiler_params=pltpu.CompilerParams(
            dimension_semantics=("arbitrary",)),
    )(u.reshape(E, 1), W2[:, 0].reshape(1, F), b2.reshape(1, F))
    return out


# speedup vs baseline: 182.6088x; 1.9834x over previous
import functools
import jax
import jax.numpy as jnp
from jax import lax
from jax.experimental import pallas as pl
from jax.experimental.pallas import tpu as pltpu
from jax.experimental.pallas import tpu_sc as plsc

L = 16
NW = 16
NP = 10240


def _make_sc_pipeline(E, n_nodes):
    EW = E // NW
    NSLICE = NP // NW
    VECS_E = EW // L
    VECS_S = NSLICE // L
    assert EW % L == 0 and EW % 8 == 0 and n_nodes <= NP

    mesh = plsc.VectorSubcoreMesh(
        core_axis_name="c", subcore_axis_name="s", num_cores=1)

    @functools.partial(
        pl.kernel,
        mesh=mesh,
        compiler_params=pltpu.CompilerParams(needs_layout_passes=False),
        out_type=jax.ShapeDtypeStruct((E,), jnp.float32),
        scratch_types=[
            pltpu.VMEM((EW,), jnp.int32),
            pltpu.VMEM((EW,), jnp.int32),
            pltpu.VMEM((EW,), jnp.float32),
            pltpu.VMEM((NP,), jnp.float32),
            pltpu.VMEM((NW, NSLICE), jnp.float32),
            pltpu.VMEM((NSLICE,), jnp.float32),
            pltpu.VMEM((NSLICE,), jnp.float32),
            pltpu.VMEM((8, L), jnp.float32),
            pltpu.VMEM_SHARED((NW, NP), jnp.float32),
            pltpu.VMEM_SHARED((NP,), jnp.float32),
        ],
    )
    def sc_pipe(ei_hbm, ea_hbm, pvec_hbm, zeros_hbm, u_hbm,
                src_v, dst_v, h_v, m_v, t16_v, s_v, binv_v, pv_v,
                part_sh, bcast_sh):
        w = lax.axis_index("s")
        base = w * EW
        nbase = w * NSLICE
        pltpu.sync_copy(ei_hbm.at[pl.ds(base, EW)], src_v)
        pltpu.sync_copy(ei_hbm.at[pl.ds(E + base, EW)], dst_v)
        pltpu.sync_copy(ea_hbm.at[pl.ds(base, EW)], h_v)
        pltpu.sync_copy(pvec_hbm, pv_v)

        ones = jnp.full((L,), 1.0, jnp.float32)

        def zero_acc():
            pltpu.sync_copy(zeros_hbm, m_v)

        def scatter(use_h):
            def body(i, carry):
                b = pl.ds(i * L, L)
                si = src_v[b]
                di = dst_v[b]
                v = h_v[b] if use_h else ones
                plsc.addupdate_scatter(m_v, [si], v)
                plsc.addupdate_scatter(m_v, [di], v)
                return carry
            lax.fori_loop(0, VECS_E, body, 0)

        def publish_reduce(is_degree_round):
            pltpu.sync_copy(m_v, part_sh.at[w])
            plsc.subcore_barrier()
            for k in range(NW):
                pltpu.sync_copy(part_sh.at[k, pl.ds(nbase, NSLICE)],
                                t16_v.at[k])

            def rbody(j, carry):
                bj = pl.ds(j * L, L)
                acc = t16_v[0, bj]
                for k in range(1, NW):
                    acc = acc + t16_v[k, bj]
                if is_degree_round:
                    binv_v[bj] = jnp.where(acc > 0.0, 1.0 / acc, 0.0)
                else:
                    s_v[bj] = acc * binv_v[bj]
                return carry
            lax.fori_loop(0, VECS_S, rbody, 0)
            plsc.subcore_barrier()

        def broadcast_m():
            pltpu.sync_copy(s_v, bcast_sh.at[pl.ds(nbase, NSLICE)])
            plsc.subcore_barrier()
            pltpu.sync_copy(bcast_sh, m_v)

        def gather_round(cw, cb, do_relu):
            def body(i, carry):
                b = pl.ds(i * L, L)
                si = src_v[b]
                di = dst_v[b]
                a = plsc.load_gather(m_v, [si])
                c = plsc.load_gather(m_v, [di])
                h = cw * (a + c) + cb
                if do_relu:
                    h = jnp.maximum(h, 0.0)
                h_v[b] = h
                return carry
            lax.fori_loop(0, VECS_E, body, 0)

        zero_acc()
        scatter(use_h=False)
        publish_reduce(is_degree_round=True)

        hw0 = pv_v[0]
        cb0 = pv_v[1]
        hw1 = pv_v[2]
        cb1 = pv_v[3]
        half = pv_v[4]
        zero = pv_v[5]

        zero_acc()
        scatter(use_h=True)
        publish_reduce(is_degree_round=False)
        broadcast_m()
        gather_round(hw0, cb0, True)

        zero_acc()
        scatter(use_h=True)
        publish_reduce(is_degree_round=False)
        broadcast_m()
        gather_round(hw1, cb1, True)

        zero_acc()
        scatter(use_h=True)
        publish_reduce(is_degree_round=False)
        broadcast_m()
        gather_round(half, zero, False)

        pltpu.sync_copy(h_v, u_hbm.at[pl.ds(base, EW)])

    return sc_pipe


def _tc_outer(u_ref, w2_ref, b2_ref, o_ref):
    u = u_ref[...]
    o_ref[...] = jnp.squeeze(u, 1)[:, :, None] * w2_ref[...] + b2_ref[...]


def kernel(x, edge_index, edge_attr, batch, W0, b0, W1, b1, W2, b2):
    E = edge_attr.shape[0]
    n_nodes = x.shape[0]
    F = W2.shape[0]

    pvec = (jnp.zeros((8, L), jnp.float32)
            .at[0].set(0.5 * W0[0, 0])
            .at[1].set(b0[0])
            .at[2].set(0.5 * W1[0, 0])
            .at[3].set(b1[0])
            .at[4].set(0.5))
    zeros = jnp.zeros((NP,), jnp.float32)

    u = _make_sc_pipeline(E, n_nodes)(edge_index.reshape(-1), edge_attr,
                                      pvec, zeros)

    R = E // 128
    BR = 20
    out3 = pl.pallas_call(
        _tc_outer,
        grid=(R // BR,),
        in_specs=[
            pl.BlockSpec((BR, 1, 128), lambda i: (i, 0, 0)),
            pl.BlockSpec((1, 1, F), lambda i: (0, 0, 0)),
            pl.BlockSpec((1, 1, F), lambda i: (0, 0, 0)),
        ],
        out_specs=pl.BlockSpec((BR, 128, F), lambda i: (i, 0, 0)),
        out_shape=jax.ShapeDtypeStruct((R, 128, F), jnp.float32),
        compiler_params=pltpu.CompilerParams(
            dimension_semantics=("arbitrary",)),
    )(u.reshape(R, 1, 128), W2[:, 0].reshape(1, 1, F), b2.reshape(1, 1, F))
    return out3.reshape(E, F)


# speedup vs baseline: 311.7622x; 1.7073x over previous
import functools
import jax
import jax.numpy as jnp
from jax import lax
from jax.experimental import pallas as pl
from jax.experimental.pallas import tpu as pltpu
from jax.experimental.pallas import tpu_sc as plsc

L = 16
NW = 16
NP = 10240


def _make_sc_pipeline(E, n_nodes):
    EW = E // NW
    NSLICE = NP // NW
    VECS_E = EW // L
    VECS_S = NSLICE // L
    assert EW % L == 0 and EW % 8 == 0 and n_nodes <= NP

    mesh = plsc.VectorSubcoreMesh(
        core_axis_name="c", subcore_axis_name="s", num_cores=1)

    @functools.partial(
        pl.kernel,
        mesh=mesh,
        compiler_params=pltpu.CompilerParams(needs_layout_passes=False),
        out_type=jax.ShapeDtypeStruct((E,), jnp.float32),
        scratch_types=[
            pltpu.VMEM((EW,), jnp.int32),
            pltpu.VMEM((EW,), jnp.int32),
            pltpu.VMEM((EW,), jnp.float32),
            pltpu.VMEM((NP,), jnp.float32),
            pltpu.VMEM((NP,), jnp.float32),
            pltpu.VMEM((NW, NSLICE), jnp.float32),
            pltpu.VMEM((NW, NSLICE), jnp.float32),
            pltpu.VMEM((NSLICE,), jnp.float32),
            pltpu.VMEM((NSLICE,), jnp.float32),
            pltpu.VMEM((8, L), jnp.float32),
            pltpu.VMEM_SHARED((NW, NP), jnp.float32),
            pltpu.VMEM_SHARED((NW, NP), jnp.float32),
            pltpu.VMEM_SHARED((NP,), jnp.float32),
            pltpu.SemaphoreType.DMA,
        ],
    )
    def sc_pipe(ei_hbm, ea_hbm, pvec_hbm, zeros_hbm, u_hbm,
                src_v, dst_v, h_v, m_v, c_v, t16_v, u16_v, s_v, binv_v, pv_v,
                part_sh, cnt_sh, bcast_sh, sem):
        w = lax.axis_index("s")
        base = w * EW
        nbase = w * NSLICE

        cps = [
            pltpu.async_copy(ei_hbm.at[pl.ds(base, EW)], src_v, sem),
            pltpu.async_copy(ei_hbm.at[pl.ds(E + base, EW)], dst_v, sem),
            pltpu.async_copy(ea_hbm.at[pl.ds(base, EW)], h_v, sem),
            pltpu.async_copy(pvec_hbm, pv_v, sem),
            pltpu.async_copy(zeros_hbm, m_v, sem),
            pltpu.async_copy(zeros_hbm, c_v, sem),
        ]
        for cp in cps:
            cp.wait()

        ones = jnp.full((L,), 1.0, jnp.float32)

        def zero_acc():
            pltpu.sync_copy(zeros_hbm, m_v)

        def scatter(with_counts):
            @plsc.parallel_loop(0, VECS_E, unroll=10)
            def _(i):
                b = pl.ds(i * L, L)
                si = src_v[b]
                di = dst_v[b]
                v = h_v[b]
                plsc.addupdate_scatter(m_v, [si], v)
                plsc.addupdate_scatter(m_v, [di], v)
                if with_counts:
                    plsc.addupdate_scatter(c_v, [si], ones)
                    plsc.addupdate_scatter(c_v, [di], ones)

        def publish_reduce(with_counts):
            pcs = [pltpu.async_copy(m_v, part_sh.at[w], sem)]
            if with_counts:
                pcs.append(pltpu.async_copy(c_v, cnt_sh.at[w], sem))
            for cp in pcs:
                cp.wait()
            plsc.subcore_barrier()
            fcs = [pltpu.async_copy(part_sh.at[k, pl.ds(nbase, NSLICE)],
                                    t16_v.at[k], sem) for k in range(NW)]
            if with_counts:
                fcs += [pltpu.async_copy(cnt_sh.at[k, pl.ds(nbase, NSLICE)],
                                         u16_v.at[k], sem) for k in range(NW)]
            for cp in fcs:
                cp.wait()

            @plsc.parallel_loop(0, VECS_S, unroll=8)
            def _(j):
                bj = pl.ds(j * L, L)
                acc = t16_v[0, bj]
                for k in range(1, NW):
                    acc = acc + t16_v[k, bj]
                if with_counts:
                    cnt = u16_v[0, bj]
                    for k in range(1, NW):
                        cnt = cnt + u16_v[k, bj]
                    binv = jnp.where(cnt > 0.0, 1.0 / cnt, 0.0)
                    binv_v[bj] = binv
                    s_v[bj] = acc * binv
                else:
                    s_v[bj] = acc * binv_v[bj]
            plsc.subcore_barrier()

        def broadcast_m():
            pltpu.sync_copy(s_v, bcast_sh.at[pl.ds(nbase, NSLICE)])
            plsc.subcore_barrier()
            pltpu.sync_copy(bcast_sh, m_v)

        def gather_round(cw, cb, do_relu):
            @plsc.parallel_loop(0, VECS_E, unroll=10)
            def _(i):
                b = pl.ds(i * L, L)
                si = src_v[b]
                di = dst_v[b]
                a = plsc.load_gather(m_v, [si])
                c = plsc.load_gather(m_v, [di])
                h = cw * (a + c) + cb
                if do_relu:
                    h = jnp.maximum(h, 0.0)
                h_v[b] = h

        hw0 = pv_v[0]
        cb0 = pv_v[1]
        hw1 = pv_v[2]
        cb1 = pv_v[3]
        half = pv_v[4]
        zero = pv_v[5]

        scatter(with_counts=True)
        publish_reduce(with_counts=True)
        broadcast_m()
        gather_round(hw0, cb0, True)

        zero_acc()
        scatter(with_counts=False)
        publish_reduce(with_counts=False)
        broadcast_m()
        gather_round(hw1, cb1, True)

        zero_acc()
        scatter(with_counts=False)
        publish_reduce(with_counts=False)
        broadcast_m()
        gather_round(half, zero, False)

        pltpu.sync_copy(h_v, u_hbm.at[pl.ds(base, EW)])

    return sc_pipe


def _tc_outer(u_ref, w2_ref, b2_ref, o_ref):
    u = u_ref[...]
    o_ref[...] = jnp.squeeze(u, 1)[:, :, None] * w2_ref[...] + b2_ref[...]


def kernel(x, edge_index, edge_attr, batch, W0, b0, W1, b1, W2, b2):
    E = edge_attr.shape[0]
    n_nodes = x.shape[0]
    F = W2.shape[0]

    pvec = (jnp.zeros((8, L), jnp.float32)
            .at[0].set(0.5 * W0[0, 0])
            .at[1].set(b0[0])
            .at[2].set(0.5 * W1[0, 0])
            .at[3].set(b1[0])
            .at[4].set(0.5))
    zeros = jnp.zeros((NP,), jnp.float32)

    u = _make_sc_pipeline(E, n_nodes)(edge_index.reshape(-1), edge_attr,
                                      pvec, zeros)

    R = E // 128
    BR = 100
    out3 = pl.pallas_call(
        _tc_outer,
        grid=(R // BR,),
        in_specs=[
            pl.BlockSpec((BR, 1, 128), lambda i: (i, 0, 0)),
            pl.BlockSpec((1, 1, F), lambda i: (0, 0, 0)),
            pl.BlockSpec((1, 1, F), lambda i: (0, 0, 0)),
        ],
        out_specs=pl.BlockSpec((BR, 128, F), lambda i: (i, 0, 0)),
        out_shape=jax.ShapeDtypeStruct((R, 128, F), jnp.float32),
        compiler_params=pltpu.CompilerParams(
            dimension_semantics=("arbitrary",)),
    )(u.reshape(R, 1, 128), W2[:, 0].reshape(1, 1, F), b2.reshape(1, 1, F))
    return out3.reshape(E, F)


# speedup vs baseline: 326.5751x; 1.0475x over previous
import functools
import jax
import jax.numpy as jnp
from jax import lax
from jax.experimental import pallas as pl
from jax.experimental.pallas import tpu as pltpu
from jax.experimental.pallas import tpu_sc as plsc

L = 16
NW = 16
NP = 10240


def _make_sc_pipeline(E, n_nodes):
    EW = E // NW
    NSLICE = NP // NW
    VECS_E = EW // L
    VECS_S = NSLICE // L
    assert EW % L == 0 and EW % 8 == 0 and n_nodes <= NP

    mesh = plsc.VectorSubcoreMesh(
        core_axis_name="c", subcore_axis_name="s", num_cores=1)

    @functools.partial(
        pl.kernel,
        mesh=mesh,
        compiler_params=pltpu.CompilerParams(needs_layout_passes=False),
        out_type=jax.ShapeDtypeStruct((E,), jnp.float32),
        scratch_types=[
            pltpu.VMEM((EW,), jnp.int32),
            pltpu.VMEM((EW,), jnp.int32),
            pltpu.VMEM((EW,), jnp.float32),
            pltpu.VMEM((NP,), jnp.float32),
            pltpu.VMEM((NP,), jnp.float32),
            pltpu.VMEM((NW, NSLICE), jnp.float32),
            pltpu.VMEM((NW, NSLICE), jnp.float32),
            pltpu.VMEM((NSLICE,), jnp.float32),
            pltpu.VMEM((NSLICE,), jnp.float32),
            pltpu.VMEM((8, L), jnp.float32),
            pltpu.VMEM_SHARED((NW, NP), jnp.float32),
            pltpu.VMEM_SHARED((NW, NP), jnp.float32),
            pltpu.VMEM_SHARED((NP,), jnp.float32),
            pltpu.SemaphoreType.DMA,
        ],
    )
    def sc_pipe(ei_hbm, ea_hbm, pvec_hbm, zeros_hbm, u_hbm,
                src_v, dst_v, h_v, m_v, c_v, t16_v, u16_v, s_v, binv_v, pv_v,
                part_sh, cnt_sh, bcast_sh, sem):
        w = lax.axis_index("s")
        base = w * EW
        nbase = w * NSLICE

        cps = [
            pltpu.async_copy(ei_hbm.at[pl.ds(base, EW)], src_v, sem),
            pltpu.async_copy(ei_hbm.at[pl.ds(E + base, EW)], dst_v, sem),
            pltpu.async_copy(ea_hbm.at[pl.ds(base, EW)], h_v, sem),
            pltpu.async_copy(pvec_hbm, pv_v, sem),
            pltpu.async_copy(zeros_hbm, m_v, sem),
            pltpu.async_copy(zeros_hbm, c_v, sem),
        ]
        for cp in cps:
            cp.wait()

        ones = jnp.full((L,), 1.0, jnp.float32)

        def scatter(with_counts):
            @plsc.parallel_loop(0, VECS_E, unroll=10)
            def _(i):
                b = pl.ds(i * L, L)
                si = src_v[b]
                di = dst_v[b]
                v = h_v[b]
                plsc.addupdate_scatter(m_v, [si], v)
                plsc.addupdate_scatter(m_v, [di], v)
                if with_counts:
                    plsc.addupdate_scatter(c_v, [si], ones)
                    plsc.addupdate_scatter(c_v, [di], ones)

        def publish_reduce(acc_v, with_counts, zero_c_after):
            pcs = [pltpu.async_copy(acc_v, part_sh.at[w], sem)]
            if with_counts:
                pcs.append(pltpu.async_copy(c_v, cnt_sh.at[w], sem))
            for cp in pcs:
                cp.wait()
            plsc.subcore_barrier()
            fcs = [pltpu.async_copy(part_sh.at[k, pl.ds(nbase, NSLICE)],
                                    t16_v.at[k], sem) for k in range(NW)]
            if with_counts:
                fcs += [pltpu.async_copy(cnt_sh.at[k, pl.ds(nbase, NSLICE)],
                                         u16_v.at[k], sem) for k in range(NW)]
            for cp in fcs:
                cp.wait()
            zc = (pltpu.async_copy(zeros_hbm, c_v, sem)
                  if zero_c_after else None)

            @plsc.parallel_loop(0, VECS_S, unroll=8)
            def _(j):
                bj = pl.ds(j * L, L)
                acc = t16_v[0, bj]
                for k in range(1, NW):
                    acc = acc + t16_v[k, bj]
                if with_counts:
                    cnt = u16_v[0, bj]
                    for k in range(1, NW):
                        cnt = cnt + u16_v[k, bj]
                    binv = jnp.where(cnt > 0.0, 1.0 / cnt, 0.0)
                    binv_v[bj] = binv
                    s_v[bj] = acc * binv
                else:
                    s_v[bj] = acc * binv_v[bj]
            if zc is not None:
                zc.wait()
            plsc.subcore_barrier()

        def broadcast_m():
            pltpu.sync_copy(s_v, bcast_sh.at[pl.ds(nbase, NSLICE)])
            plsc.subcore_barrier()
            pltpu.sync_copy(bcast_sh, m_v)

        def fused_gather_scatter(cw, cb):
            @plsc.parallel_loop(0, VECS_E, unroll=10)
            def _(i):
                b = pl.ds(i * L, L)
                si = src_v[b]
                di = dst_v[b]
                a = plsc.load_gather(m_v, [si])
                c = plsc.load_gather(m_v, [di])
                h = jnp.maximum(cw * (a + c) + cb, 0.0)
                plsc.addupdate_scatter(c_v, [si], h)
                plsc.addupdate_scatter(c_v, [di], h)

        def final_gather(cw):
            @plsc.parallel_loop(0, VECS_E, unroll=10)
            def _(i):
                b = pl.ds(i * L, L)
                si = src_v[b]
                di = dst_v[b]
                a = plsc.load_gather(m_v, [si])
                c = plsc.load_gather(m_v, [di])
                h_v[b] = cw * (a + c)

        hw0 = pv_v[0]
        cb0 = pv_v[1]
        hw1 = pv_v[2]
        cb1 = pv_v[3]
        half = pv_v[4]

        scatter(with_counts=True)
        publish_reduce(m_v, with_counts=True, zero_c_after=True)
        broadcast_m()
        fused_gather_scatter(hw0, cb0)
        publish_reduce(c_v, with_counts=False, zero_c_after=True)
        broadcast_m()
        fused_gather_scatter(hw1, cb1)
        publish_reduce(c_v, with_counts=False, zero_c_after=False)
        broadcast_m()
        final_gather(half)

        pltpu.sync_copy(h_v, u_hbm.at[pl.ds(base, EW)])

    return sc_pipe


def _tc_outer(u_ref, w2_ref, b2_ref, o_ref):
    u = u_ref[...]
    o_ref[...] = jnp.squeeze(u, 1)[:, :, None] * w2_ref[...] + b2_ref[...]


def kernel(x, edge_index, edge_attr, batch, W0, b0, W1, b1, W2, b2):
    E = edge_attr.shape[0]
    n_nodes = x.shape[0]
    F = W2.shape[0]

    pvec = (jnp.zeros((8, L), jnp.float32)
            .at[0].set(0.5 * W0[0, 0])
            .at[1].set(b0[0])
            .at[2].set(0.5 * W1[0, 0])
            .at[3].set(b1[0])
            .at[4].set(0.5))
    zeros = jnp.zeros((NP,), jnp.float32)

    u = _make_sc_pipeline(E, n_nodes)(edge_index.reshape(-1), edge_attr,
                                      pvec, zeros)

    R = E // 128
    BR = 100
    out3 = pl.pallas_call(
        _tc_outer,
        grid=(R // BR,),
        in_specs=[
            pl.BlockSpec((BR, 1, 128), lambda i: (i, 0, 0)),
            pl.BlockSpec((1, 1, F), lambda i: (0, 0, 0)),
            pl.BlockSpec((1, 1, F), lambda i: (0, 0, 0)),
        ],
        out_specs=pl.BlockSpec((BR, 128, F), lambda i: (i, 0, 0)),
        out_shape=jax.ShapeDtypeStruct((R, 128, F), jnp.float32),
        compiler_params=pltpu.CompilerParams(
            dimension_semantics=("arbitrary",)),
    )(u.reshape(R, 1, 128), W2[:, 0].reshape(1, 1, F), b2.reshape(1, 1, F))
    return out3.reshape(E, F)


# speedup vs baseline: 331.9664x; 1.0165x over previous
import functools
import jax
import jax.numpy as jnp
from jax import lax
from jax.experimental import pallas as pl
from jax.experimental.pallas import tpu as pltpu
from jax.experimental.pallas import tpu_sc as plsc

L = 16
NW = 16
NP = 10240


def _make_sc_pipeline(E, n_nodes):
    EW = E // NW
    NSLICE = NP // NW
    VECS_E = EW // L
    VECS_S = NSLICE // L
    assert EW % L == 0 and EW % 8 == 0 and n_nodes <= NP

    mesh = plsc.VectorSubcoreMesh(
        core_axis_name="c", subcore_axis_name="s", num_cores=1)

    @functools.partial(
        pl.kernel,
        mesh=mesh,
        compiler_params=pltpu.CompilerParams(needs_layout_passes=False),
        out_type=jax.ShapeDtypeStruct((E,), jnp.float32),
        scratch_types=[
            pltpu.VMEM((EW,), jnp.int32),
            pltpu.VMEM((EW,), jnp.int32),
            pltpu.VMEM((EW,), jnp.float32),
            pltpu.VMEM((NP,), jnp.float32),
            pltpu.VMEM((NP,), jnp.float32),
            pltpu.VMEM((NW, NSLICE), jnp.float32),
            pltpu.VMEM((NW, NSLICE), jnp.float32),
            pltpu.VMEM((NSLICE,), jnp.float32),
            pltpu.VMEM((NSLICE,), jnp.float32),
            pltpu.VMEM((8, L), jnp.float32),
            pltpu.VMEM_SHARED((NW, NP), jnp.float32),
            pltpu.VMEM_SHARED((NW, NP), jnp.float32),
            pltpu.VMEM_SHARED((NP,), jnp.float32),
            pltpu.SemaphoreType.DMA,
        ],
    )
    def sc_pipe(ei_hbm, ea_hbm, pvec_hbm, zeros_hbm, u_hbm,
                src_v, dst_v, h_v, m_v, c_v, t16_v, u16_v, s_v, binv_v, pv_v,
                part_sh, cnt_sh, bcast_sh, sem):
        w = lax.axis_index("s")
        base = w * EW
        nbase = w * NSLICE

        cps = [
            pltpu.async_copy(ei_hbm.at[pl.ds(base, EW)], src_v, sem),
            pltpu.async_copy(ei_hbm.at[pl.ds(E + base, EW)], dst_v, sem),
            pltpu.async_copy(ea_hbm.at[pl.ds(base, EW)], h_v, sem),
            pltpu.async_copy(pvec_hbm, pv_v, sem),
            pltpu.async_copy(zeros_hbm, m_v, sem),
            pltpu.async_copy(zeros_hbm, c_v, sem),
        ]
        for cp in cps:
            cp.wait()

        ones = jnp.full((L,), 1.0, jnp.float32)

        def scatter(with_counts):
            @plsc.parallel_loop(0, VECS_E, unroll=10)
            def _(i):
                b = pl.ds(i * L, L)
                si = src_v[b]
                di = dst_v[b]
                v = h_v[b]
                plsc.addupdate_scatter(m_v, [si], v)
                plsc.addupdate_scatter(m_v, [di], v)
                if with_counts:
                    plsc.addupdate_scatter(c_v, [si], ones)
                    plsc.addupdate_scatter(c_v, [di], ones)

        def publish_reduce(acc_v, with_counts, zero_c_after):
            pcs = [pltpu.async_copy(acc_v, part_sh.at[w], sem)]
            if with_counts:
                pcs.append(pltpu.async_copy(c_v, cnt_sh.at[w], sem))
            for cp in pcs:
                cp.wait()
            plsc.subcore_barrier()
            fcs = [pltpu.async_copy(part_sh.at[k, pl.ds(nbase, NSLICE)],
                                    t16_v.at[k], sem) for k in range(NW)]
            if with_counts:
                fcs += [pltpu.async_copy(cnt_sh.at[k, pl.ds(nbase, NSLICE)],
                                         u16_v.at[k], sem) for k in range(NW)]
            for cp in fcs:
                cp.wait()
            zc = (pltpu.async_copy(zeros_hbm, c_v, sem)
                  if zero_c_after else None)

            @plsc.parallel_loop(0, VECS_S, unroll=8)
            def _(j):
                bj = pl.ds(j * L, L)
                acc = t16_v[0, bj]
                for k in range(1, NW):
                    acc = acc + t16_v[k, bj]
                if with_counts:
                    cnt = u16_v[0, bj]
                    for k in range(1, NW):
                        cnt = cnt + u16_v[k, bj]
                    binv = jnp.where(cnt > 0.0, 1.0 / cnt, 0.0)
                    binv_v[bj] = binv
                    s_v[bj] = acc * binv
                else:
                    s_v[bj] = acc * binv_v[bj]
            if zc is not None:
                zc.wait()
            plsc.subcore_barrier()

        def broadcast_m():
            pltpu.sync_copy(s_v, bcast_sh.at[pl.ds(nbase, NSLICE)])
            plsc.subcore_barrier()
            pltpu.sync_copy(bcast_sh, m_v)

        def fused_gather_scatter(cw, cb):
            @plsc.parallel_loop(0, VECS_E, unroll=10)
            def _(i):
                b = pl.ds(i * L, L)
                si = src_v[b]
                di = dst_v[b]
                a = plsc.load_gather(m_v, [si])
                c = plsc.load_gather(m_v, [di])
                h = jnp.maximum(cw * (a + c) + cb, 0.0)
                plsc.addupdate_scatter(c_v, [si], h)
                plsc.addupdate_scatter(c_v, [di], h)

        def final_gather(cw):
            @plsc.parallel_loop(0, VECS_E, unroll=10)
            def _(i):
                b = pl.ds(i * L, L)
                si = src_v[b]
                di = dst_v[b]
                a = plsc.load_gather(m_v, [si])
                c = plsc.load_gather(m_v, [di])
                h_v[b] = cw * (a + c)

        hw0 = pv_v[0]
        cb0 = pv_v[1]
        hw1 = pv_v[2]
        cb1 = pv_v[3]
        half = pv_v[4]

        scatter(with_counts=True)
        publish_reduce(m_v, with_counts=True, zero_c_after=True)
        broadcast_m()
        fused_gather_scatter(hw0, cb0)
        publish_reduce(c_v, with_counts=False, zero_c_after=True)
        broadcast_m()
        fused_gather_scatter(hw1, cb1)
        publish_reduce(c_v, with_counts=False, zero_c_after=False)
        broadcast_m()
        final_gather(half)

        pltpu.sync_copy(h_v, u_hbm.at[pl.ds(base, EW)])

    return sc_pipe


def _tc_outer(u_ref, w2_ref, b2_ref, o_ref):
    u = u_ref[...]
    o_ref[...] = jnp.squeeze(u, 1)[:, :, None] * w2_ref[...] + b2_ref[...]


def kernel(x, edge_index, edge_attr, batch, W0, b0, W1, b1, W2, b2):
    E = edge_attr.shape[0]
    n_nodes = x.shape[0]
    F = W2.shape[0]

    pvec = (jnp.zeros((8, L), jnp.float32)
            .at[0].set(0.5 * W0[0, 0])
            .at[1].set(b0[0])
            .at[2].set(0.5 * W1[0, 0])
            .at[3].set(b1[0])
            .at[4].set(0.5))
    zeros = jnp.zeros((NP,), jnp.float32)

    u = _make_sc_pipeline(E, n_nodes)(edge_index.reshape(-1), edge_attr,
                                      pvec, zeros)

    R = E // 128
    BR = 250
    out3 = pl.pallas_call(
        _tc_outer,
        grid=(R // BR,),
        in_specs=[
            pl.BlockSpec((BR, 1, 128), lambda i: (i, 0, 0)),
            pl.BlockSpec((1, 1, F), lambda i: (0, 0, 0)),
            pl.BlockSpec((1, 1, F), lambda i: (0, 0, 0)),
        ],
        out_specs=pl.BlockSpec((BR, 128, F), lambda i: (i, 0, 0)),
        out_shape=jax.ShapeDtypeStruct((R, 128, F), jnp.float32),
        compiler_params=pltpu.CompilerParams(
            dimension_semantics=("arbitrary",)),
    )(u.reshape(R, 1, 128), W2[:, 0].reshape(1, 1, F), b2.reshape(1, 1, F))
    return out3.reshape(E, F)
